# initial kernel scaffold (unmeasured)
import jax
import jax.numpy as jnp
from jax import lax
from jax.experimental import pallas as pl
from jax.experimental.pallas import tpu as pltpu


def kernel(
    t,
):
    def body(*refs):
        pass

    out_shape = jax.ShapeDtypeStruct(..., jnp.float32)
    return pl.pallas_call(body, out_shape=out_shape)(...)


# baseline (device time: 1192925 ns/iter reference)
import jax
import jax.numpy as jnp
from jax import lax
from jax.experimental import pallas as pl
from jax.experimental.pallas import tpu as pltpu

N_DEV = 4
T_SUB = 2


def kernel(t):
    m_per, n = t.shape
    qr = m_per // N_DEV
    r = qr // T_SUB
    n_hops = T_SUB * 2 * (N_DEV - 1)

    def body(t_hbm, out_hbm, acc, recv, local,
             send_sem, recv_sem, load_sem, store_sem, credit_sem):
        my = lax.axis_index("i")
        left = lax.rem(my - 1 + N_DEV, N_DEV)
        right = lax.rem(my + 1, N_DEV)
        q_mine = lax.rem(my + 1, N_DEV)

        barrier = pltpu.get_barrier_semaphore()
        for nbr in (left, right):
            pl.semaphore_signal(barrier, inc=1, device_id=(nbr,),
                                device_id_type=pl.DeviceIdType.MESH)
        pl.semaphore_wait(barrier, 2)

        pl.semaphore_signal(credit_sem, inc=1, device_id=(left,),
                            device_id_type=pl.DeviceIdType.MESH)

        def ring_hop(src_ref, dst_slot):
            pl.semaphore_wait(credit_sem, 1)
            rdma = pltpu.make_async_remote_copy(
                src_ref=src_ref, dst_ref=recv.at[dst_slot],
                send_sem=send_sem, recv_sem=recv_sem,
                device_id=(right,), device_id_type=pl.DeviceIdType.MESH)
            rdma.start()
            return rdma

        def load(row_start, dst):
            cp = pltpu.make_async_copy(
                t_hbm.at[pl.ds(row_start, r), :], dst, load_sem)
            cp.start()
            return cp

        def store(src, row_start):
            cp = pltpu.make_async_copy(
                src, out_hbm.at[pl.ds(row_start, r), :], store_sem)
            cp.start()
            cp.wait()

        hop = 0
        for j in range(T_SUB):
            load(my * qr + j * r, acc).wait()

            for s in range(N_DEV - 1):
                rdma = ring_hop(acc, 0)
                q = lax.rem(my - s - 1 + N_DEV, N_DEV)
                cp = load(q * qr + j * r, local)
                cp.wait()
                rdma.wait()
                acc[:, :] = recv[0] + local[:, :]
                hop += 1
                if hop < n_hops:
                    pl.semaphore_signal(credit_sem, inc=1, device_id=(left,),
                                        device_id_type=pl.DeviceIdType.MESH)

            s_val = acc[:, :]
            relu = jnp.maximum(s_val, 0.0)
            acc[:, :] = jnp.tanh(s_val) * s_val * s_val + relu * relu * relu
            store(acc, q_mine * qr + j * r)

            for h in range(N_DEV - 1):
                src = acc if h == 0 else recv.at[(h - 1) % 2]
                rdma = ring_hop(src, h % 2)
                rdma.wait()
                q = lax.rem(my - h + N_DEV, N_DEV)
                store(recv.at[h % 2], q * qr + j * r)
                hop += 1
                if hop < n_hops:
                    pl.semaphore_signal(credit_sem, inc=1, device_id=(left,),
                                        device_id_type=pl.DeviceIdType.MESH)

    return pl.pallas_call(
        body,
        out_shape=jax.ShapeDtypeStruct((m_per, n), jnp.float32),
        in_specs=[pl.BlockSpec(memory_space=pl.ANY)],
        out_specs=pl.BlockSpec(memory_space=pl.ANY),
        scratch_shapes=[
            pltpu.VMEM((r, n), jnp.float32),
            pltpu.VMEM((2, r, n), jnp.float32),
            pltpu.VMEM((r, n), jnp.float32),
            pltpu.SemaphoreType.DMA,
            pltpu.SemaphoreType.DMA,
            pltpu.SemaphoreType.DMA,
            pltpu.SemaphoreType.DMA,
            pltpu.SemaphoreType.REGULAR,
        ],
        compiler_params=pltpu.CompilerParams(collective_id=0),
    )(t)


# device time: 596542 ns/iter; 1.9997x vs baseline; 1.9997x over previous
import jax
import jax.numpy as jnp
from jax import lax
from jax.experimental import pallas as pl
from jax.experimental.pallas import tpu as pltpu

N_DEV = 4
T_SUB = 2
N_SLOTS = 3
H = 2 * (N_DEV - 1) * T_SUB


def kernel(t):
    m_per, n = t.shape
    qr = m_per // N_DEV
    r = qr // T_SUB
    half = n // 2

    def body(t_hbm, out_hbm,
             acc_cw, acc_ccw, recv_cw, recv_ccw, loc_cw, loc_ccw,
             send_sems_cw, send_sems_ccw, recv_sems_cw, recv_sems_ccw,
             load_sem_cw, load_sem_ccw, store_sem_cw, store_sem_ccw,
             credit_cw, credit_ccw):
        my = lax.axis_index("i")
        left = lax.rem(my + N_DEV - 1, N_DEV)
        right = lax.rem(my + 1, N_DEV)

        barrier = pltpu.get_barrier_semaphore()
        for nbr in (left, right):
            pl.semaphore_signal(barrier, inc=1, device_id=(nbr,),
                                device_id_type=pl.DeviceIdType.MESH)
        pl.semaphore_wait(barrier, 2)

        pl.semaphore_signal(credit_cw, inc=2, device_id=(left,),
                            device_id_type=pl.DeviceIdType.MESH)
        pl.semaphore_signal(credit_ccw, inc=2, device_id=(right,),
                            device_id_type=pl.DeviceIdType.MESH)

        dirs = {
            "cw": dict(acc=acc_cw, recv=recv_cw, loc=loc_cw,
                       ssem=send_sems_cw, rsem=recv_sems_cw,
                       lsem=load_sem_cw, osem=store_sem_cw,
                       credit=credit_cw, dst=right, upstream=left,
                       col0=0),
            "ccw": dict(acc=acc_ccw, recv=recv_ccw, loc=loc_ccw,
                        ssem=send_sems_ccw, rsem=recv_sems_ccw,
                        lsem=load_sem_ccw, osem=store_sem_ccw,
                        credit=credit_ccw, dst=left, upstream=right,
                        col0=half),
        }
        def q_of(offset):
            return lax.rem(my + offset + 2 * N_DEV, N_DEV)

        sign = {"cw": -1, "ccw": +1}

        descs = {"cw": [None] * H, "ccw": [None] * H}

        def issue_send(d, k):
            D = dirs[d]
            j = k % T_SUB
            pl.semaphore_wait(D["credit"], 1)
            rdma = pltpu.make_async_remote_copy(
                src_ref=D["acc"].at[j],
                dst_ref=D["recv"].at[k % N_SLOTS],
                send_sem=D["ssem"].at[k % 2],
                recv_sem=D["rsem"].at[k % N_SLOTS],
                device_id=(D["dst"],),
                device_id_type=pl.DeviceIdType.MESH)
            rdma.start()
            descs[d][k] = rdma

        def issue_load(d, k):
            D = dirs[d]
            s, j = k // T_SUB, k % T_SUB
            q = q_of(sign[d] * (s + 1))
            cp = pltpu.make_async_copy(
                t_hbm.at[pl.ds(q * qr + j * r, r),
                         pl.ds(D["col0"], half)],
                D["loc"], D["lsem"])
            cp.start()
            return cp

        def store(d, src, q, j):
            D = dirs[d]
            cp = pltpu.make_async_copy(
                src, out_hbm.at[pl.ds(q * qr + j * r, r),
                                pl.ds(D["col0"], half)], D["osem"])
            cp.start()
            cp.wait()

        init_cps = []
        for d in ("cw", "ccw"):
            D = dirs[d]
            for j in range(T_SUB):
                cp = pltpu.make_async_copy(
                    t_hbm.at[pl.ds(my * qr + j * r, r),
                             pl.ds(D["col0"], half)],
                    D["acc"].at[j], D["lsem"])
                cp.start()
                init_cps.append(cp)
        for cp in init_cps:
            cp.wait()
        issue_send("cw", 0)
        issue_send("ccw", 0)

        for h in range(H):
            p, j = h // T_SUB, h % T_SUB
            if h + 1 < H:
                issue_send("cw", h + 1)
                issue_send("ccw", h + 1)
            loads = {}
            if p <= N_DEV - 2:
                loads["cw"] = issue_load("cw", h)
                loads["ccw"] = issue_load("ccw", h)

            for d in ("cw", "ccw"):
                D = dirs[d]
                rdma = descs[d][h]
                rdma.wait_recv()
                slot = h % N_SLOTS
                if p <= N_DEV - 2:
                    s = p
                    loads[d].wait()
                    rdma.wait_send()
                    s_val = D["recv"][slot] + D["loc"][:, :]
                    if s == N_DEV - 2:
                        relu = jnp.maximum(s_val, 0.0)
                        s_val = (jnp.tanh(s_val) * s_val * s_val
                                 + relu * relu * relu)
                        D["acc"][j] = s_val
                        store(d, D["acc"].at[j], q_of(-sign[d]), j)
                    else:
                        D["acc"][j] = s_val
                else:
                    hh = p - (N_DEV - 1)
                    rdma.wait_send()
                    store(d, D["recv"].at[slot], q_of(sign[d] * hh), j)
                    if hh < N_DEV - 2:
                        D["acc"][j] = D["recv"][slot]
                if h < H - 2:
                    pl.semaphore_signal(
                        D["credit"], inc=1, device_id=(D["upstream"],),
                        device_id_type=pl.DeviceIdType.MESH)

    return pl.pallas_call(
        body,
        out_shape=jax.ShapeDtypeStruct((m_per, n), jnp.float32),
        in_specs=[pl.BlockSpec(memory_space=pl.ANY)],
        out_specs=pl.BlockSpec(memory_space=pl.ANY),
        scratch_shapes=[
            pltpu.VMEM((T_SUB, r, half), jnp.float32),
            pltpu.VMEM((T_SUB, r, half), jnp.float32),
            pltpu.VMEM((N_SLOTS, r, half), jnp.float32),
            pltpu.VMEM((N_SLOTS, r, half), jnp.float32),
            pltpu.VMEM((r, half), jnp.float32),
            pltpu.VMEM((r, half), jnp.float32),
            pltpu.SemaphoreType.DMA((2,)),
            pltpu.SemaphoreType.DMA((2,)),
            pltpu.SemaphoreType.DMA((N_SLOTS,)),
            pltpu.SemaphoreType.DMA((N_SLOTS,)),
            pltpu.SemaphoreType.DMA,
            pltpu.SemaphoreType.DMA,
            pltpu.SemaphoreType.DMA,
            pltpu.SemaphoreType.DMA,
            pltpu.SemaphoreType.REGULAR,
            pltpu.SemaphoreType.REGULAR,
        ],
        compiler_params=pltpu.CompilerParams(
            collective_id=0, vmem_limit_bytes=60 * 1024 * 1024),
    )(t)


# device time: 330213 ns/iter; 3.6126x vs baseline; 1.8065x over previous
import jax
import jax.numpy as jnp
from jax import lax
from jax.experimental import pallas as pl
from jax.experimental.pallas import tpu as pltpu

N_DEV = 4
T_SUB = 2
N_SLOTS = 3
H = 2 * (N_DEV - 1) * T_SUB


def kernel(t):
    m_per, n = t.shape
    qr = m_per // N_DEV
    r = qr // T_SUB
    half = n // 2

    def body(t_hbm, out_hbm,
             acc_cw, acc_ccw, recv_cw, recv_ccw, loc_cw, loc_ccw,
             stage_cw, stage_ccw,
             send_sems_cw, send_sems_ccw, recv_sems_cw, recv_sems_ccw,
             load_sem_cw, load_sem_ccw, store_sem_cw, store_sem_ccw,
             credit_cw, credit_ccw):
        my = lax.axis_index("i")
        left = lax.rem(my + N_DEV - 1, N_DEV)
        right = lax.rem(my + 1, N_DEV)

        barrier = pltpu.get_barrier_semaphore()
        for nbr in (left, right):
            pl.semaphore_signal(barrier, inc=1, device_id=(nbr,),
                                device_id_type=pl.DeviceIdType.MESH)
        pl.semaphore_wait(barrier, 2)

        pl.semaphore_signal(credit_cw, inc=2, device_id=(left,),
                            device_id_type=pl.DeviceIdType.MESH)
        pl.semaphore_signal(credit_ccw, inc=2, device_id=(right,),
                            device_id_type=pl.DeviceIdType.MESH)

        dirs = {
            "cw": dict(acc=acc_cw, recv=recv_cw, loc=loc_cw,
                       stage=stage_cw,
                       ssem=send_sems_cw, rsem=recv_sems_cw,
                       lsem=load_sem_cw, osem=store_sem_cw,
                       credit=credit_cw, dst=right, upstream=left,
                       col0=0),
            "ccw": dict(acc=acc_ccw, recv=recv_ccw, loc=loc_ccw,
                        stage=stage_ccw,
                        ssem=send_sems_ccw, rsem=recv_sems_ccw,
                        lsem=load_sem_ccw, osem=store_sem_ccw,
                        credit=credit_ccw, dst=left, upstream=right,
                        col0=half),
        }
        def q_of(offset):
            return lax.rem(my + offset + 2 * N_DEV, N_DEV)

        sign = {"cw": -1, "ccw": +1}

        descs = {"cw": [None] * H, "ccw": [None] * H}

        def issue_send(d, k):
            D = dirs[d]
            j = k % T_SUB
            pl.semaphore_wait(D["credit"], 1)
            rdma = pltpu.make_async_remote_copy(
                src_ref=D["acc"].at[j],
                dst_ref=D["recv"].at[k % N_SLOTS],
                send_sem=D["ssem"].at[k % 2],
                recv_sem=D["rsem"].at[k % N_SLOTS],
                device_id=(D["dst"],),
                device_id_type=pl.DeviceIdType.MESH)
            rdma.start()
            descs[d][k] = rdma

        def issue_load(d, k):
            D = dirs[d]
            s, j = k // T_SUB, k % T_SUB
            q = q_of(sign[d] * (s + 1))
            cp = pltpu.make_async_copy(
                t_hbm.at[pl.ds(q * qr + j * r, r),
                         pl.ds(D["col0"], half)],
                D["loc"], D["lsem"])
            cp.start()
            return cp

        def store(d, src, q, j):
            D = dirs[d]
            cp = pltpu.make_async_copy(
                src, out_hbm.at[pl.ds(q * qr + j * r, r),
                                pl.ds(D["col0"], half)], D["osem"])
            cp.start()
            cp.wait()

        for j in range(T_SUB):
            cps = []
            for d in ("cw", "ccw"):
                D = dirs[d]
                dst = D["loc"] if j == 0 else D["stage"]
                cp = pltpu.make_async_copy(
                    t_hbm.at[pl.ds(my * qr + j * r, r),
                             pl.ds(D["col0"], half)], dst, D["lsem"])
                cp.start()
                cps.append((D, dst, cp))
            for D, dst, cp in cps:
                cp.wait()
                D["acc"][j] = dst[:, :].astype(jnp.bfloat16)
        issue_send("cw", 0)
        issue_send("ccw", 0)

        for h in range(H):
            p, j = h // T_SUB, h % T_SUB
            if h + 1 < H:
                issue_send("cw", h + 1)
                issue_send("ccw", h + 1)
            loads = {}
            if p <= N_DEV - 2:
                loads["cw"] = issue_load("cw", h)
                loads["ccw"] = issue_load("ccw", h)

            for d in ("cw", "ccw"):
                D = dirs[d]
                rdma = descs[d][h]
                rdma.wait_recv()
                slot = h % N_SLOTS
                if p <= N_DEV - 2:
                    s = p
                    loads[d].wait()
                    rdma.wait_send()
                    s_val = (D["recv"][slot].astype(jnp.float32)
                             + D["loc"][:, :])
                    if s == N_DEV - 2:
                        relu = jnp.maximum(s_val, 0.0)
                        s_val = (jnp.tanh(s_val) * s_val * s_val
                                 + relu * relu * relu)
                        D["stage"][:, :] = s_val
                        D["acc"][j] = s_val.astype(jnp.bfloat16)
                        store(d, D["stage"], q_of(-sign[d]), j)
                    else:
                        D["acc"][j] = s_val.astype(jnp.bfloat16)
                else:
                    hh = p - (N_DEV - 1)
                    rdma.wait_send()
                    D["stage"][:, :] = D["recv"][slot].astype(jnp.float32)
                    store(d, D["stage"], q_of(sign[d] * hh), j)
                    if hh < N_DEV - 2:
                        D["acc"][j] = D["recv"][slot]
                if h < H - 2:
                    pl.semaphore_signal(
                        D["credit"], inc=1, device_id=(D["upstream"],),
                        device_id_type=pl.DeviceIdType.MESH)

    return pl.pallas_call(
        body,
        out_shape=jax.ShapeDtypeStruct((m_per, n), jnp.float32),
        in_specs=[pl.BlockSpec(memory_space=pl.ANY)],
        out_specs=pl.BlockSpec(memory_space=pl.ANY),
        scratch_shapes=[
            pltpu.VMEM((T_SUB, r, half), jnp.bfloat16),
            pltpu.VMEM((T_SUB, r, half), jnp.bfloat16),
            pltpu.VMEM((N_SLOTS, r, half), jnp.bfloat16),
            pltpu.VMEM((N_SLOTS, r, half), jnp.bfloat16),
            pltpu.VMEM((r, half), jnp.float32),
            pltpu.VMEM((r, half), jnp.float32),
            pltpu.VMEM((r, half), jnp.float32),
            pltpu.VMEM((r, half), jnp.float32),
            pltpu.SemaphoreType.DMA((2,)),
            pltpu.SemaphoreType.DMA((2,)),
            pltpu.SemaphoreType.DMA((N_SLOTS,)),
            pltpu.SemaphoreType.DMA((N_SLOTS,)),
            pltpu.SemaphoreType.DMA,
            pltpu.SemaphoreType.DMA,
            pltpu.SemaphoreType.DMA,
            pltpu.SemaphoreType.DMA,
            pltpu.SemaphoreType.REGULAR,
            pltpu.SemaphoreType.REGULAR,
        ],
        compiler_params=pltpu.CompilerParams(
            collective_id=0, vmem_limit_bytes=60 * 1024 * 1024),
    )(t)


# device time: 329264 ns/iter; 3.6230x vs baseline; 1.0029x over previous
import jax
import jax.numpy as jnp
from jax import lax
from jax.experimental import pallas as pl
from jax.experimental.pallas import tpu as pltpu

N_DEV = 4
T_SUB = 2
N_SLOTS = 3
H = 2 * (N_DEV - 1) * T_SUB


def kernel(t):
    m_per, n = t.shape
    qr = m_per // N_DEV
    r = qr // T_SUB
    half = n // 2

    def body(t_hbm, out_hbm,
             acc_cw, acc_ccw, recv_cw, recv_ccw, loc_cw, loc_ccw,
             stage_cw, stage_ccw,
             send_sems_cw, send_sems_ccw, recv_sems_cw, recv_sems_ccw,
             load_sem_cw, load_sem_ccw, store_sem_cw, store_sem_ccw,
             credit_cw, credit_ccw):
        my = lax.axis_index("i")
        left = lax.rem(my + N_DEV - 1, N_DEV)
        right = lax.rem(my + 1, N_DEV)

        barrier = pltpu.get_barrier_semaphore()
        for nbr in (left, right):
            pl.semaphore_signal(barrier, inc=1, device_id=(nbr,),
                                device_id_type=pl.DeviceIdType.MESH)
        pl.semaphore_wait(barrier, 2)

        pl.semaphore_signal(credit_cw, inc=2, device_id=(left,),
                            device_id_type=pl.DeviceIdType.MESH)
        pl.semaphore_signal(credit_ccw, inc=2, device_id=(right,),
                            device_id_type=pl.DeviceIdType.MESH)

        dirs = {
            "cw": dict(acc=acc_cw, recv=recv_cw, loc=loc_cw,
                       stage=stage_cw,
                       ssem=send_sems_cw, rsem=recv_sems_cw,
                       lsem=load_sem_cw, osem=store_sem_cw,
                       credit=credit_cw, dst=right, upstream=left,
                       col0=0),
            "ccw": dict(acc=acc_ccw, recv=recv_ccw, loc=loc_ccw,
                        stage=stage_ccw,
                        ssem=send_sems_ccw, rsem=recv_sems_ccw,
                        lsem=load_sem_ccw, osem=store_sem_ccw,
                        credit=credit_ccw, dst=left, upstream=right,
                        col0=half),
        }
        def q_of(offset):
            return lax.rem(my + offset + 2 * N_DEV, N_DEV)

        sign = {"cw": -1, "ccw": +1}

        descs = {"cw": [None] * H, "ccw": [None] * H}

        def issue_send(d, k):
            D = dirs[d]
            j = k % T_SUB
            pl.semaphore_wait(D["credit"], 1)
            rdma = pltpu.make_async_remote_copy(
                src_ref=D["acc"].at[j],
                dst_ref=D["recv"].at[k % N_SLOTS],
                send_sem=D["ssem"].at[k % 2],
                recv_sem=D["rsem"].at[k % N_SLOTS],
                device_id=(D["dst"],),
                device_id_type=pl.DeviceIdType.MESH)
            rdma.start()
            descs[d][k] = rdma

        def issue_load(d, k):
            D = dirs[d]
            s, j = k // T_SUB, k % T_SUB
            q = q_of(sign[d] * (s + 1))
            cp = pltpu.make_async_copy(
                t_hbm.at[pl.ds(q * qr + j * r, r),
                         pl.ds(D["col0"], half)],
                D["loc"], D["lsem"])
            cp.start()
            return cp

        pending = {"cw": [None, None], "ccw": [None, None]}

        def stage_ready(d, sl):
            if pending[d][sl] is not None:
                pending[d][sl].wait()
                pending[d][sl] = None

        def store(d, sl, q, j):
            D = dirs[d]
            cp = pltpu.make_async_copy(
                D["stage"].at[sl],
                out_hbm.at[pl.ds(q * qr + j * r, r),
                           pl.ds(D["col0"], half)], D["osem"])
            cp.start()
            pending[d][sl] = cp

        for j in range(T_SUB):
            cps = []
            for d in ("cw", "ccw"):
                D = dirs[d]
                dst = D["loc"] if j == 0 else D["stage"].at[0]
                cp = pltpu.make_async_copy(
                    t_hbm.at[pl.ds(my * qr + j * r, r),
                             pl.ds(D["col0"], half)], dst, D["lsem"])
                cp.start()
                cps.append((D, dst, cp))
            for D, dst, cp in cps:
                cp.wait()
                D["acc"][j] = dst[:, :].astype(jnp.bfloat16)
        issue_send("cw", 0)
        issue_send("ccw", 0)

        for h in range(H):
            p, j = h // T_SUB, h % T_SUB
            if h + 1 < H:
                issue_send("cw", h + 1)
                issue_send("ccw", h + 1)
            loads = {}
            if p <= N_DEV - 2:
                loads["cw"] = issue_load("cw", h)
                loads["ccw"] = issue_load("ccw", h)

            for d in ("cw", "ccw"):
                D = dirs[d]
                rdma = descs[d][h]
                rdma.wait_recv()
                slot = h % N_SLOTS
                if p <= N_DEV - 2:
                    s = p
                    loads[d].wait()
                    rdma.wait_send()
                    s_val = (D["recv"][slot].astype(jnp.float32)
                             + D["loc"][:, :])
                    if s == N_DEV - 2:
                        relu = jnp.maximum(s_val, 0.0)
                        s_val = (jnp.tanh(s_val) * s_val * s_val
                                 + relu * relu * relu)
                        stage_ready(d, h % 2)
                        D["stage"][h % 2] = s_val
                        D["acc"][j] = s_val.astype(jnp.bfloat16)
                        store(d, h % 2, q_of(-sign[d]), j)
                    else:
                        D["acc"][j] = s_val.astype(jnp.bfloat16)
                else:
                    hh = p - (N_DEV - 1)
                    rdma.wait_send()
                    stage_ready(d, h % 2)
                    D["stage"][h % 2] = D["recv"][slot].astype(jnp.float32)
                    store(d, h % 2, q_of(sign[d] * hh), j)
                    if hh < N_DEV - 2:
                        D["acc"][j] = D["recv"][slot]
                if h < H - 2:
                    pl.semaphore_signal(
                        D["credit"], inc=1, device_id=(D["upstream"],),
                        device_id_type=pl.DeviceIdType.MESH)

        for d in ("cw", "ccw"):
            for sl in range(2):
                stage_ready(d, sl)

    return pl.pallas_call(
        body,
        out_shape=jax.ShapeDtypeStruct((m_per, n), jnp.float32),
        in_specs=[pl.BlockSpec(memory_space=pl.ANY)],
        out_specs=pl.BlockSpec(memory_space=pl.ANY),
        scratch_shapes=[
            pltpu.VMEM((T_SUB, r, half), jnp.bfloat16),
            pltpu.VMEM((T_SUB, r, half), jnp.bfloat16),
            pltpu.VMEM((N_SLOTS, r, half), jnp.bfloat16),
            pltpu.VMEM((N_SLOTS, r, half), jnp.bfloat16),
            pltpu.VMEM((r, half), jnp.float32),
            pltpu.VMEM((r, half), jnp.float32),
            pltpu.VMEM((2, r, half), jnp.float32),
            pltpu.VMEM((2, r, half), jnp.float32),
            pltpu.SemaphoreType.DMA((2,)),
            pltpu.SemaphoreType.DMA((2,)),
            pltpu.SemaphoreType.DMA((N_SLOTS,)),
            pltpu.SemaphoreType.DMA((N_SLOTS,)),
            pltpu.SemaphoreType.DMA,
            pltpu.SemaphoreType.DMA,
            pltpu.SemaphoreType.DMA,
            pltpu.SemaphoreType.DMA,
            pltpu.SemaphoreType.REGULAR,
            pltpu.SemaphoreType.REGULAR,
        ],
        compiler_params=pltpu.CompilerParams(
            collective_id=0, vmem_limit_bytes=60 * 1024 * 1024),
    )(t)


# device time: 324615 ns/iter; 3.6749x vs baseline; 1.0143x over previous
import jax
import jax.numpy as jnp
from jax import lax
from jax.experimental import pallas as pl
from jax.experimental.pallas import tpu as pltpu

N_DEV = 4
T_SUB = 2
N_SLOTS = 4
N_CREDITS = N_SLOTS - 1
H = 2 * (N_DEV - 1) * T_SUB


def kernel(t):
    m_per, n = t.shape
    qr = m_per // N_DEV
    r = qr // T_SUB
    half = n // 2

    def body(t_hbm, out_hbm,
             acc_cw, acc_ccw, recv_cw, recv_ccw, loc_cw, loc_ccw,
             stage_cw, stage_ccw,
             send_sems_cw, send_sems_ccw, recv_sems_cw, recv_sems_ccw,
             load_sem_cw, load_sem_ccw, store_sem_cw, store_sem_ccw,
             credit_cw, credit_ccw):
        my = lax.axis_index("i")
        left = lax.rem(my + N_DEV - 1, N_DEV)
        right = lax.rem(my + 1, N_DEV)

        dirs = {
            "cw": dict(acc=acc_cw, recv=recv_cw, loc=loc_cw,
                       stage=stage_cw,
                       ssem=send_sems_cw, rsem=recv_sems_cw,
                       lsem=load_sem_cw, osem=store_sem_cw,
                       credit=credit_cw, dst=right, upstream=left,
                       col0=0),
            "ccw": dict(acc=acc_ccw, recv=recv_ccw, loc=loc_ccw,
                        stage=stage_ccw,
                        ssem=send_sems_ccw, rsem=recv_sems_ccw,
                        lsem=load_sem_ccw, osem=store_sem_ccw,
                        credit=credit_ccw, dst=left, upstream=right,
                        col0=half),
        }
        def q_of(offset):
            return lax.rem(my + offset + 2 * N_DEV, N_DEV)

        sign = {"cw": -1, "ccw": +1}

        descs = {"cw": [None] * H, "ccw": [None] * H}

        def issue_send(d, k):
            D = dirs[d]
            j = k % T_SUB
            pl.semaphore_wait(D["credit"], 1)
            rdma = pltpu.make_async_remote_copy(
                src_ref=D["acc"].at[j],
                dst_ref=D["recv"].at[k % N_SLOTS],
                send_sem=D["ssem"].at[k % 2],
                recv_sem=D["rsem"].at[k % N_SLOTS],
                device_id=(D["dst"],),
                device_id_type=pl.DeviceIdType.MESH)
            rdma.start()
            descs[d][k] = rdma

        def issue_load(d, k):
            D = dirs[d]
            s, j = k // T_SUB, k % T_SUB
            q = q_of(sign[d] * (s + 1))
            cp = pltpu.make_async_copy(
                t_hbm.at[pl.ds(q * qr + j * r, r),
                         pl.ds(D["col0"], half)],
                D["loc"], D["lsem"])
            cp.start()
            return cp

        pending = {"cw": [None, None], "ccw": [None, None]}

        def stage_ready(d, sl):
            if pending[d][sl] is not None:
                pending[d][sl].wait()
                pending[d][sl] = None

        def store(d, sl, q, j):
            D = dirs[d]
            cp = pltpu.make_async_copy(
                D["stage"].at[sl],
                out_hbm.at[pl.ds(q * qr + j * r, r),
                           pl.ds(D["col0"], half)], D["osem"])
            cp.start()
            pending[d][sl] = cp

        init_cps = {}
        for j in range(T_SUB):
            for d in ("cw", "ccw"):
                D = dirs[d]
                dst = D["loc"] if j == 0 else D["stage"].at[0]
                sem = D["lsem"] if j == 0 else D["osem"]
                cp = pltpu.make_async_copy(
                    t_hbm.at[pl.ds(my * qr + j * r, r),
                             pl.ds(D["col0"], half)], dst, sem)
                cp.start()
                init_cps[(d, j)] = (dst, cp)

        barrier = pltpu.get_barrier_semaphore()
        for nbr in (left, right):
            pl.semaphore_signal(barrier, inc=1, device_id=(nbr,),
                                device_id_type=pl.DeviceIdType.MESH)
        pl.semaphore_wait(barrier, 2)

        pl.semaphore_signal(credit_cw, inc=N_CREDITS, device_id=(left,),
                            device_id_type=pl.DeviceIdType.MESH)
        pl.semaphore_signal(credit_ccw, inc=N_CREDITS, device_id=(right,),
                            device_id_type=pl.DeviceIdType.MESH)

        for j in range(T_SUB):
            for d in ("cw", "ccw"):
                dst, cp = init_cps[(d, j)]
                cp.wait()
                dirs[d]["acc"][j] = dst[:, :].astype(jnp.bfloat16)
            if j == 0:
                issue_send("cw", 0)
                issue_send("ccw", 0)

        for h in range(H):
            p, j = h // T_SUB, h % T_SUB
            if h + 1 < H:
                issue_send("cw", h + 1)
                issue_send("ccw", h + 1)
            loads = {}
            if p <= N_DEV - 2:
                loads["cw"] = issue_load("cw", h)
                loads["ccw"] = issue_load("ccw", h)

            for d in ("cw", "ccw"):
                D = dirs[d]
                rdma = descs[d][h]
                rdma.wait_recv()
                slot = h % N_SLOTS
                if p <= N_DEV - 2:
                    s = p
                    loads[d].wait()
                    rdma.wait_send()
                    s_val = (D["recv"][slot].astype(jnp.float32)
                             + D["loc"][:, :])
                    if s == N_DEV - 2:
                        relu = jnp.maximum(s_val, 0.0)
                        s_val = (jnp.tanh(s_val) * s_val * s_val
                                 + relu * relu * relu)
                        stage_ready(d, h % 2)
                        D["stage"][h % 2] = s_val
                        D["acc"][j] = s_val.astype(jnp.bfloat16)
                        store(d, h % 2, q_of(-sign[d]), j)
                    else:
                        D["acc"][j] = s_val.astype(jnp.bfloat16)
                else:
                    hh = p - (N_DEV - 1)
                    rdma.wait_send()
                    stage_ready(d, h % 2)
                    D["stage"][h % 2] = D["recv"][slot].astype(jnp.float32)
                    store(d, h % 2, q_of(sign[d] * hh), j)
                    if hh < N_DEV - 2:
                        D["acc"][j] = D["recv"][slot]
                if h < H - N_CREDITS:
                    pl.semaphore_signal(
                        D["credit"], inc=1, device_id=(D["upstream"],),
                        device_id_type=pl.DeviceIdType.MESH)

        for d in ("cw", "ccw"):
            for sl in range(2):
                stage_ready(d, sl)

    return pl.pallas_call(
        body,
        out_shape=jax.ShapeDtypeStruct((m_per, n), jnp.float32),
        in_specs=[pl.BlockSpec(memory_space=pl.ANY)],
        out_specs=pl.BlockSpec(memory_space=pl.ANY),
        scratch_shapes=[
            pltpu.VMEM((T_SUB, r, half), jnp.bfloat16),
            pltpu.VMEM((T_SUB, r, half), jnp.bfloat16),
            pltpu.VMEM((N_SLOTS, r, half), jnp.bfloat16),
            pltpu.VMEM((N_SLOTS, r, half), jnp.bfloat16),
            pltpu.VMEM((r, half), jnp.float32),
            pltpu.VMEM((r, half), jnp.float32),
            pltpu.VMEM((2, r, half), jnp.float32),
            pltpu.VMEM((2, r, half), jnp.float32),
            pltpu.SemaphoreType.DMA((2,)),
            pltpu.SemaphoreType.DMA((2,)),
            pltpu.SemaphoreType.DMA((N_SLOTS,)),
            pltpu.SemaphoreType.DMA((N_SLOTS,)),
            pltpu.SemaphoreType.DMA,
            pltpu.SemaphoreType.DMA,
            pltpu.SemaphoreType.DMA,
            pltpu.SemaphoreType.DMA,
            pltpu.SemaphoreType.REGULAR,
            pltpu.SemaphoreType.REGULAR,
        ],
        compiler_params=pltpu.CompilerParams(
            collective_id=0, vmem_limit_bytes=60 * 1024 * 1024),
    )(t)


# device time: 323088 ns/iter; 3.6923x vs baseline; 1.0047x over previous
import jax
import jax.numpy as jnp
from jax import lax
from jax.experimental import pallas as pl
from jax.experimental.pallas import tpu as pltpu

N_DEV = 4
T_SUB = 4
N_SLOTS = 4
N_CREDITS = N_SLOTS - 1
H = 2 * (N_DEV - 1) * T_SUB


def kernel(t):
    m_per, n = t.shape
    qr = m_per // N_DEV
    r = qr // T_SUB
    half = n // 2

    def body(t_hbm, out_hbm,
             acc_cw, acc_ccw, recv_cw, recv_ccw, loc_cw, loc_ccw,
             stage_cw, stage_ccw,
             send_sems_cw, send_sems_ccw, recv_sems_cw, recv_sems_ccw,
             load_sem_cw, load_sem_ccw, store_sem_cw, store_sem_ccw,
             credit_cw, credit_ccw):
        my = lax.axis_index("i")
        left = lax.rem(my + N_DEV - 1, N_DEV)
        right = lax.rem(my + 1, N_DEV)

        dirs = {
            "cw": dict(acc=acc_cw, recv=recv_cw, loc=loc_cw,
                       stage=stage_cw,
                       ssem=send_sems_cw, rsem=recv_sems_cw,
                       lsem=load_sem_cw, osem=store_sem_cw,
                       credit=credit_cw, dst=right, upstream=left,
                       col0=0),
            "ccw": dict(acc=acc_ccw, recv=recv_ccw, loc=loc_ccw,
                        stage=stage_ccw,
                        ssem=send_sems_ccw, rsem=recv_sems_ccw,
                        lsem=load_sem_ccw, osem=store_sem_ccw,
                        credit=credit_ccw, dst=left, upstream=right,
                        col0=half),
        }
        def q_of(offset):
            return lax.rem(my + offset + 2 * N_DEV, N_DEV)

        sign = {"cw": -1, "ccw": +1}

        descs = {"cw": [None] * H, "ccw": [None] * H}

        def issue_send(d, k):
            D = dirs[d]
            j = k % T_SUB
            pl.semaphore_wait(D["credit"], 1)
            rdma = pltpu.make_async_remote_copy(
                src_ref=D["acc"].at[j],
                dst_ref=D["recv"].at[k % N_SLOTS],
                send_sem=D["ssem"].at[k % 2],
                recv_sem=D["rsem"].at[k % N_SLOTS],
                device_id=(D["dst"],),
                device_id_type=pl.DeviceIdType.MESH)
            rdma.start()
            descs[d][k] = rdma

        def issue_load(d, k):
            D = dirs[d]
            s, j = k // T_SUB, k % T_SUB
            q = q_of(sign[d] * (s + 1))
            cp = pltpu.make_async_copy(
                t_hbm.at[pl.ds(q * qr + j * r, r),
                         pl.ds(D["col0"], half)],
                D["loc"], D["lsem"])
            cp.start()
            return cp

        pending = {"cw": [None, None], "ccw": [None, None]}

        def stage_ready(d, sl):
            if pending[d][sl] is not None:
                pending[d][sl].wait()
                pending[d][sl] = None

        def store(d, sl, q, j):
            D = dirs[d]
            cp = pltpu.make_async_copy(
                D["stage"].at[sl],
                out_hbm.at[pl.ds(q * qr + j * r, r),
                           pl.ds(D["col0"], half)], D["osem"])
            cp.start()
            pending[d][sl] = cp

        def start_init_pair(jbase):
            cps = {}
            for d in ("cw", "ccw"):
                D = dirs[d]
                for jo, dst, sem in ((0, D["loc"], D["lsem"]),
                                     (1, D["stage"].at[0], D["osem"])):
                    j = jbase + jo
                    cp = pltpu.make_async_copy(
                        t_hbm.at[pl.ds(my * qr + j * r, r),
                                 pl.ds(D["col0"], half)], dst, sem)
                    cp.start()
                    cps[(d, j)] = (dst, cp)
            return cps

        init_cps = start_init_pair(0)

        barrier = pltpu.get_barrier_semaphore()
        for nbr in (left, right):
            pl.semaphore_signal(barrier, inc=1, device_id=(nbr,),
                                device_id_type=pl.DeviceIdType.MESH)
        pl.semaphore_wait(barrier, 2)

        pl.semaphore_signal(credit_cw, inc=N_CREDITS, device_id=(left,),
                            device_id_type=pl.DeviceIdType.MESH)
        pl.semaphore_signal(credit_ccw, inc=N_CREDITS, device_id=(right,),
                            device_id_type=pl.DeviceIdType.MESH)

        for jbase in range(0, T_SUB, 2):
            for j in (jbase, jbase + 1):
                for d in ("cw", "ccw"):
                    dst, cp = init_cps[(d, j)]
                    cp.wait()
                    dirs[d]["acc"][j] = dst[:, :].astype(jnp.bfloat16)
            if jbase == 0:
                issue_send("cw", 0)
                issue_send("ccw", 0)
                if T_SUB > 2:
                    init_cps = start_init_pair(2)

        for h in range(H):
            p, j = h // T_SUB, h % T_SUB
            if h + 1 < H:
                issue_send("cw", h + 1)
                issue_send("ccw", h + 1)
            loads = {}
            if p <= N_DEV - 2:
                loads["cw"] = issue_load("cw", h)
                loads["ccw"] = issue_load("ccw", h)

            for d in ("cw", "ccw"):
                D = dirs[d]
                rdma = descs[d][h]
                rdma.wait_recv()
                slot = h % N_SLOTS
                if p <= N_DEV - 2:
                    s = p
                    loads[d].wait()
                    rdma.wait_send()
                    s_val = (D["recv"][slot].astype(jnp.float32)
                             + D["loc"][:, :])
                    if s == N_DEV - 2:
                        relu = jnp.maximum(s_val, 0.0)
                        s_val = (jnp.tanh(s_val) * s_val * s_val
                                 + relu * relu * relu)
                        stage_ready(d, h % 2)
                        D["stage"][h % 2] = s_val
                        D["acc"][j] = s_val.astype(jnp.bfloat16)
                        store(d, h % 2, q_of(-sign[d]), j)
                    else:
                        D["acc"][j] = s_val.astype(jnp.bfloat16)
                else:
                    hh = p - (N_DEV - 1)
                    rdma.wait_send()
                    stage_ready(d, h % 2)
                    D["stage"][h % 2] = D["recv"][slot].astype(jnp.float32)
                    store(d, h % 2, q_of(sign[d] * hh), j)
                    if hh < N_DEV - 2:
                        D["acc"][j] = D["recv"][slot]
                if h < H - N_CREDITS:
                    pl.semaphore_signal(
                        D["credit"], inc=1, device_id=(D["upstream"],),
                        device_id_type=pl.DeviceIdType.MESH)

        for d in ("cw", "ccw"):
            for sl in range(2):
                stage_ready(d, sl)

    return pl.pallas_call(
        body,
        out_shape=jax.ShapeDtypeStruct((m_per, n), jnp.float32),
        in_specs=[pl.BlockSpec(memory_space=pl.ANY)],
        out_specs=pl.BlockSpec(memory_space=pl.ANY),
        scratch_shapes=[
            pltpu.VMEM((T_SUB, r, half), jnp.bfloat16),
            pltpu.VMEM((T_SUB, r, half), jnp.bfloat16),
            pltpu.VMEM((N_SLOTS, r, half), jnp.bfloat16),
            pltpu.VMEM((N_SLOTS, r, half), jnp.bfloat16),
            pltpu.VMEM((r, half), jnp.float32),
            pltpu.VMEM((r, half), jnp.float32),
            pltpu.VMEM((2, r, half), jnp.float32),
            pltpu.VMEM((2, r, half), jnp.float32),
            pltpu.SemaphoreType.DMA((2,)),
            pltpu.SemaphoreType.DMA((2,)),
            pltpu.SemaphoreType.DMA((N_SLOTS,)),
            pltpu.SemaphoreType.DMA((N_SLOTS,)),
            pltpu.SemaphoreType.DMA,
            pltpu.SemaphoreType.DMA,
            pltpu.SemaphoreType.DMA,
            pltpu.SemaphoreType.DMA,
            pltpu.SemaphoreType.REGULAR,
            pltpu.SemaphoreType.REGULAR,
        ],
        compiler_params=pltpu.CompilerParams(
            collective_id=0, vmem_limit_bytes=60 * 1024 * 1024),
    )(t)
